# baseline (speedup 1.0000x reference)
import jax
import jax.numpy as jnp
from jax.experimental import pallas as pl
from jax.experimental.pallas import tpu as pltpu
from jax.experimental.pallas import tpu_sc as plsc

B = 16384
V = 1000000
T = 128
D = 64
H = 128

_GATHER_WINDOW = 128
_BM = 1024


def _sc_gather(table, idx):
    idx2 = idx.reshape(1, B)
    mesh = plsc.VectorSubcoreMesh(core_axis_name="core",
                                  subcore_axis_name="subcore")

    @pl.kernel(out_type=jax.ShapeDtypeStruct((B, D), table.dtype), mesh=mesh,
               compiler_params=pltpu.CompilerParams(use_tc_tiling_on_sc=False))
    def gather_kernel(tab_hbm, i_hbm, o_hbm):
        def body(i_vmem, o_vmem):
            pltpu.sync_copy(tab_hbm.at[i_vmem.at[0]], o_vmem)

        pltpu.emit_pipeline(
            body,
            grid=(B // _GATHER_WINDOW,),
            in_specs=[pl.BlockSpec((1, _GATHER_WINDOW),
                                   index_map=lambda i: (0, i))],
            out_specs=[pl.BlockSpec((_GATHER_WINDOW, D),
                                    index_map=lambda i: (i, 0))],
            core_axis_name=("core", "subcore"),
            dimension_semantics=(pltpu.PARALLEL,),
        )(i_hbm, o_hbm)

    return gather_kernel(table, idx2)


def _dense_body(id_ref, tx_ref, wp_ref, bp_ref, wg_ref, bg_ref,
                w1_ref, b1_ref, g_ref, be_ref, w2_ref, b2_ref, o_ref):
    ids = id_ref[...]
    tx = tx_ref[...]
    f32 = jnp.float32

    tp = jax.lax.dot_general(tx, wp_ref[...], (((1,), (1,)), ((), ())),
                             preferred_element_type=f32) + bp_ref[...]
    wg = wg_ref[...]
    glog = (jax.lax.dot_general(ids, wg[:, :D], (((1,), (1,)), ((), ())),
                                preferred_element_type=f32)
            + jax.lax.dot_general(tp, wg[:, D:], (((1,), (1,)), ((), ())),
                                  preferred_element_type=f32)
            + bg_ref[...])
    gate = jax.nn.sigmoid(glog)
    fused = gate * ids + (1.0 - gate) * tp

    h = jax.lax.dot_general(fused, w1_ref[...], (((1,), (1,)), ((), ())),
                            preferred_element_type=f32) + b1_ref[...]
    mu = jnp.mean(h, axis=-1, keepdims=True)
    var = jnp.mean((h - mu) ** 2, axis=-1, keepdims=True)
    h = (h - mu) * jax.lax.rsqrt(var + 1e-5) * g_ref[...] + be_ref[...]
    h = jnp.maximum(h, 0.0)

    out = jax.lax.dot_general(h, w2_ref[...], (((1,), (1,)), ((), ())),
                              preferred_element_type=f32) + b2_ref[...]
    nrm = jnp.maximum(jnp.sqrt(jnp.sum(out * out, axis=-1, keepdims=True)),
                      1e-12)
    o_ref[...] = out / nrm


def _tc_dense(id_embeds, text_feat, Wp, bp, Wg, bg, W1, b1, ln_g, ln_b,
              W2, b2):
    full = lambda shape: pl.BlockSpec(shape, lambda i: (0, 0))
    return pl.pallas_call(
        _dense_body,
        grid=(B // _BM,),
        in_specs=[
            pl.BlockSpec((_BM, D), lambda i: (i, 0)),
            pl.BlockSpec((_BM, T), lambda i: (i, 0)),
            full((D, T)),
            full((1, D)),
            full((1, 2 * D)),
            full((1, 1)),
            full((H, D)),
            full((1, H)),
            full((1, H)),
            full((1, H)),
            full((H, H)),
            full((1, H)),
        ],
        out_specs=pl.BlockSpec((_BM, H), lambda i: (i, 0)),
        out_shape=jax.ShapeDtypeStruct((B, H), jnp.float32),
    )(id_embeds, text_feat, Wp, bp.reshape(1, D), Wg, bg.reshape(1, 1),
      W1, b1.reshape(1, H), ln_g.reshape(1, H), ln_b.reshape(1, H),
      W2, b2.reshape(1, H))


@jax.jit
def kernel(text_feat, item_ids, table, Wp, bp, Wg, bg, W1, b1, ln_g, ln_b,
           W2, b2):
    idx = item_ids.astype(jnp.int32)
    id_embeds = _sc_gather(table, idx)
    return _tc_dense(id_embeds, text_feat, Wp, bp, Wg, bg, W1, b1,
                     ln_g, ln_b, W2, b2)


# speedup vs baseline: 1.0319x; 1.0319x over previous
import jax
import jax.numpy as jnp
from jax import lax
from jax.experimental import pallas as pl
from jax.experimental.pallas import tpu as pltpu
from jax.experimental.pallas import tpu_sc as plsc

B = 16384
V = 1000000
T = 128
D = 64
H = 128

_N_CHUNKS = 32
_CHUNK = B // _N_CHUNKS

_BM = 1024


def _sc_gather(table, idx):
    idx2 = idx.reshape(1, B)
    mesh = plsc.VectorSubcoreMesh(core_axis_name="core",
                                  subcore_axis_name="subcore")

    @pl.kernel(out_type=jax.ShapeDtypeStruct((B, D), table.dtype), mesh=mesh,
               scratch_types=[pltpu.VMEM((_CHUNK,), jnp.int32),
                              pltpu.SemaphoreType.DMA,
                              pltpu.SemaphoreType.DMA])
    def gather_kernel(tab_hbm, i_hbm, o_hbm, idx_vmem, isem, dsem):
        core = lax.axis_index("core")
        sub = lax.axis_index("subcore")
        chunk = core * 16 + sub
        base = chunk * _CHUNK

        pltpu.async_copy(i_hbm.at[0, pl.ds(base, _CHUNK)], idx_vmem,
                         isem).wait()

        @pl.loop(0, _CHUNK // 16)
        def _(k):
            v = idx_vmem[pl.ds(k * 16, 16)]
            for j in range(16):
                i = v[j]
                pltpu.make_async_copy(
                    tab_hbm.at[pl.ds(i, 1), :],
                    o_hbm.at[pl.ds(base + k * 16 + j, 1), :],
                    dsem,
                ).start()

        @pl.loop(0, _CHUNK)
        def _(r):
            pltpu.make_async_copy(
                tab_hbm.at[pl.ds(0, 1), :],
                o_hbm.at[pl.ds(base, 1), :],
                dsem,
            ).wait()

    return gather_kernel(table, idx2)


def _dense_body(id_ref, tx_ref, wp_ref, bp_ref, wg_ref, bg_ref,
                w1_ref, b1_ref, g_ref, be_ref, w2_ref, b2_ref, o_ref):
    ids = id_ref[...]
    tx = tx_ref[...]
    f32 = jnp.float32

    tp = jax.lax.dot_general(tx, wp_ref[...], (((1,), (1,)), ((), ())),
                             preferred_element_type=f32) + bp_ref[...]
    wg = wg_ref[...]
    glog = (jax.lax.dot_general(ids, wg[:, :D], (((1,), (1,)), ((), ())),
                                preferred_element_type=f32)
            + jax.lax.dot_general(tp, wg[:, D:], (((1,), (1,)), ((), ())),
                                  preferred_element_type=f32)
            + bg_ref[...])
    gate = jax.nn.sigmoid(glog)
    fused = gate * ids + (1.0 - gate) * tp

    h = jax.lax.dot_general(fused, w1_ref[...], (((1,), (1,)), ((), ())),
                            preferred_element_type=f32) + b1_ref[...]
    mu = jnp.mean(h, axis=-1, keepdims=True)
    var = jnp.mean((h - mu) ** 2, axis=-1, keepdims=True)
    h = (h - mu) * jax.lax.rsqrt(var + 1e-5) * g_ref[...] + be_ref[...]
    h = jnp.maximum(h, 0.0)

    out = jax.lax.dot_general(h, w2_ref[...], (((1,), (1,)), ((), ())),
                              preferred_element_type=f32) + b2_ref[...]
    nrm = jnp.maximum(jnp.sqrt(jnp.sum(out * out, axis=-1, keepdims=True)),
                      1e-12)
    o_ref[...] = out / nrm


def _tc_dense(id_embeds, text_feat, Wp, bp, Wg, bg, W1, b1, ln_g, ln_b,
              W2, b2):
    full = lambda shape: pl.BlockSpec(shape, lambda i: (0,) * len(shape))
    return pl.pallas_call(
        _dense_body,
        grid=(B // _BM,),
        in_specs=[
            pl.BlockSpec((_BM, D), lambda i: (i, 0)),
            pl.BlockSpec((_BM, T), lambda i: (i, 0)),
            full((D, T)),
            full((1, D)),
            full((1, 2 * D)),
            full((1, 1)),
            full((H, D)),
            full((1, H)),
            full((1, H)),
            full((1, H)),
            full((H, H)),
            full((1, H)),
        ],
        out_specs=pl.BlockSpec((_BM, H), lambda i: (i, 0)),
        out_shape=jax.ShapeDtypeStruct((B, H), jnp.float32),
    )(id_embeds, text_feat, Wp, bp.reshape(1, D), Wg, bg.reshape(1, 1),
      W1, b1.reshape(1, H), ln_g.reshape(1, H), ln_b.reshape(1, H),
      W2, b2.reshape(1, H))


@jax.jit
def kernel(text_feat, item_ids, table, Wp, bp, Wg, bg, W1, b1, ln_g, ln_b,
           W2, b2):
    idx = item_ids.astype(jnp.int32)
    id_embeds = _sc_gather(table, idx)
    return _tc_dense(id_embeds, text_feat, Wp, bp, Wg, bg, W1, b1,
                     ln_g, ln_b, W2, b2)


# speedup vs baseline: 2.3113x; 2.2399x over previous
import jax
import jax.numpy as jnp
from jax.experimental import pallas as pl
from jax.experimental.pallas import tpu as pltpu
from jax.experimental.pallas import tpu_sc as plsc

B = 16384
V = 1000000
T = 128
D = 64
H = 128

_BN = 16384
_HB = _BN // 2
_NBLK = -(-V // _BN)
_PR = _NBLK * _HB
_GATHER_WINDOW = 128
_BM = 1024


def _transpose_body(t_ref, o_ref):
    x = t_ref[...]
    y1 = jnp.transpose(x[:, :_HB], (1, 0))
    y2 = jnp.transpose(x[:, _HB:], (1, 0))
    o_ref[...] = jnp.concatenate([y1, y2], axis=1)


def _tc_transpose(tabT):
    return pl.pallas_call(
        _transpose_body,
        grid=(_NBLK,),
        in_specs=[pl.BlockSpec((D, _BN), lambda i: (0, i))],
        out_specs=pl.BlockSpec((_HB, 2 * D), lambda i: (i, 0)),
        out_shape=jax.ShapeDtypeStruct((_PR, 2 * D), jnp.float32),
        compiler_params=pltpu.CompilerParams(
            dimension_semantics=("arbitrary",)),
    )(tabT)


def _sc_gather(pairs, pidx):
    idx2 = pidx.reshape(1, B)
    mesh = plsc.VectorSubcoreMesh(core_axis_name="core",
                                  subcore_axis_name="subcore")

    @pl.kernel(out_type=jax.ShapeDtypeStruct((B, 2 * D), pairs.dtype),
               mesh=mesh)
    def gather_kernel(tab_hbm, i_hbm, o_hbm):
        def body(i_vmem, o_vmem):
            pltpu.sync_copy(tab_hbm.at[i_vmem.at[0]], o_vmem)

        pltpu.emit_pipeline(
            body,
            grid=(B // _GATHER_WINDOW,),
            in_specs=[pl.BlockSpec((1, _GATHER_WINDOW),
                                   index_map=lambda i: (0, i))],
            out_specs=[pl.BlockSpec((_GATHER_WINDOW, 2 * D),
                                    index_map=lambda i: (i, 0))],
            core_axis_name=("core", "subcore"),
            dimension_semantics=(pltpu.PARALLEL,),
        )(i_hbm, o_hbm)

    return gather_kernel(pairs, idx2)


def _dense_body(pr_ref, par_ref, tx_ref, wp_ref, bp_ref, wg_ref, bg_ref,
                w1_ref, b1_ref, g_ref, be_ref, w2_ref, b2_ref, o_ref):
    pr = pr_ref[...]
    par = par_ref[...]
    tx = tx_ref[...]
    f32 = jnp.float32

    ids = pr[:, :D] * (1.0 - par) + pr[:, D:] * par

    tp = jax.lax.dot_general(tx, wp_ref[...], (((1,), (1,)), ((), ())),
                             preferred_element_type=f32) + bp_ref[...]
    wg = wg_ref[...]
    glog = (jax.lax.dot_general(ids, wg[:, :D], (((1,), (1,)), ((), ())),
                                preferred_element_type=f32)
            + jax.lax.dot_general(tp, wg[:, D:], (((1,), (1,)), ((), ())),
                                  preferred_element_type=f32)
            + bg_ref[...])
    gate = jax.nn.sigmoid(glog)
    fused = gate * ids + (1.0 - gate) * tp

    h = jax.lax.dot_general(fused, w1_ref[...], (((1,), (1,)), ((), ())),
                            preferred_element_type=f32) + b1_ref[...]
    mu = jnp.mean(h, axis=-1, keepdims=True)
    var = jnp.mean((h - mu) ** 2, axis=-1, keepdims=True)
    h = (h - mu) * jax.lax.rsqrt(var + 1e-5) * g_ref[...] + be_ref[...]
    h = jnp.maximum(h, 0.0)

    out = jax.lax.dot_general(h, w2_ref[...], (((1,), (1,)), ((), ())),
                              preferred_element_type=f32) + b2_ref[...]
    nrm = jnp.maximum(jnp.sqrt(jnp.sum(out * out, axis=-1, keepdims=True)),
                      1e-12)
    o_ref[...] = out / nrm


def _tc_dense(pairs_g, par_f, text_feat, Wp, bp, Wg, bg, W1, b1, ln_g, ln_b,
              W2, b2):
    full = lambda shape: pl.BlockSpec(shape, lambda i: (0,) * len(shape))
    return pl.pallas_call(
        _dense_body,
        grid=(B // _BM,),
        in_specs=[
            pl.BlockSpec((_BM, 2 * D), lambda i: (i, 0)),
            pl.BlockSpec((_BM, 1), lambda i: (i, 0)),
            pl.BlockSpec((_BM, T), lambda i: (i, 0)),
            full((D, T)),
            full((1, D)),
            full((1, 2 * D)),
            full((1, 1)),
            full((H, D)),
            full((1, H)),
            full((1, H)),
            full((1, H)),
            full((H, H)),
            full((1, H)),
        ],
        out_specs=pl.BlockSpec((_BM, H), lambda i: (i, 0)),
        out_shape=jax.ShapeDtypeStruct((B, H), jnp.float32),
    )(pairs_g, par_f, text_feat, Wp, bp.reshape(1, D), Wg, bg.reshape(1, 1),
      W1, b1.reshape(1, H), ln_g.reshape(1, H), ln_b.reshape(1, H),
      W2, b2.reshape(1, H))


@jax.jit
def kernel(text_feat, item_ids, table, Wp, bp, Wg, bg, W1, b1, ln_g, ln_b,
           W2, b2):
    idx = item_ids.astype(jnp.int32)
    pairs = _tc_transpose(table.T)
    blk = idx // _BN
    off = idx - blk * _BN
    half = off // _HB
    pidx = blk * _HB + (off - half * _HB)
    pairs_g = _sc_gather(pairs, pidx)
    par_f = half.astype(jnp.float32).reshape(B, 1)
    return _tc_dense(pairs_g, par_f, text_feat, Wp, bp, Wg, bg, W1, b1,
                     ln_g, ln_b, W2, b2)
